# Initial kernel scaffold; baseline (speedup 1.0000x reference)
import jax
import jax.numpy as jnp
from jax.experimental import pallas as pl


def kernel(tokenized_text, embedded_text, image_embeds, placeholder_token, Wq1, Wk1, Wv1, Wo1, bo1, Wq2, Wk2, Wv2, Wo2, bo2, Wn, bn):
    raise NotImplementedError("write your pallas kernel here")


# speedup vs baseline: 1.2175x; 1.2175x over previous
import jax
import jax.numpy as jnp
from jax.experimental import pallas as pl
from jax.experimental.pallas import tpu as pltpu


def _fused_body(ph_ref, tok_ref, emb_ref, x_ref, wv_ref, wo_ref, bo_ref,
                wn_ref, bn_ref, out_ref):
    x = x_ref[...]
    t = jnp.dot(x, wv_ref[...], preferred_element_type=jnp.float32)
    t = jnp.dot(t, wo_ref[...], preferred_element_type=jnp.float32) + bo_ref[...]
    p = jnp.dot(t, wn_ref[...], preferred_element_type=jnp.float32) + bn_ref[...]
    mask = tok_ref[...] == ph_ref[0]
    out_ref[...] = jnp.where(mask, p, emb_ref[...])


def kernel(tokenized_text, embedded_text, image_embeds, placeholder_token,
           Wq1, Wk1, Wv1, Wo1, bo1, Wq2, Wk2, Wv2, Wo2, bo2, Wn, bn):
    b, n = tokenized_text.shape
    d = embedded_text.shape[-1]
    tok = tokenized_text.reshape(n, 1)
    emb = embedded_text.reshape(n, d)
    x = image_embeds.reshape(1, d)
    ph = placeholder_token.reshape(1)
    out = pl.pallas_call(
        _fused_body,
        out_shape=jax.ShapeDtypeStruct((n, d), jnp.float32),
        in_specs=[
            pl.BlockSpec(memory_space=pltpu.SMEM),
            pl.BlockSpec(memory_space=pltpu.VMEM),
            pl.BlockSpec(memory_space=pltpu.VMEM),
            pl.BlockSpec(memory_space=pltpu.VMEM),
            pl.BlockSpec(memory_space=pltpu.VMEM),
            pl.BlockSpec(memory_space=pltpu.VMEM),
            pl.BlockSpec(memory_space=pltpu.VMEM),
            pl.BlockSpec(memory_space=pltpu.VMEM),
            pl.BlockSpec(memory_space=pltpu.VMEM),
        ],
        out_specs=pl.BlockSpec(memory_space=pltpu.VMEM),
    )(ph, tok, emb, x, Wv2, Wo2, bo2.reshape(1, d), Wn, bn.reshape(1, d))
    return out.reshape(b, n, d)
